# baseline (speedup 1.0000x reference)
import functools

import jax
import jax.numpy as jnp
from jax import lax
from jax.experimental import pallas as pl
from jax.experimental.pallas import tpu as pltpu
from jax.experimental.pallas import tpu_sc as plsc

N = 10000
E = 320000
D = 128
EA = 16

NC = 2
NS = 16
NW = NC * NS
EPW = E // NW
K = 80
CH = EPW // K

ZR = 40
NZCH = N // ZR

BE = 1000
BN = 1000


def _sc_gather(x, col2):
    mesh = plsc.VectorSubcoreMesh(core_axis_name="c", subcore_axis_name="s")

    @functools.partial(
        pl.kernel,
        out_type=jax.ShapeDtypeStruct((E, D), jnp.float32),
        mesh=mesh,
        scratch_types=[
            pltpu.VMEM((K,), jnp.int32),
            pltpu.VMEM((K, D), jnp.float32),
            pltpu.SemaphoreType.DMA,
        ],
    )
    def gather_kernel(x_hbm, col_hbm, out_hbm, idx_v, rows_v, sem):
        wid = lax.axis_index("c") * NS + lax.axis_index("s")
        base = wid * EPW

        def body(j, carry):
            pltpu.sync_copy(col_hbm.at[wid * CH + j], idx_v)
            pltpu.async_copy(x_hbm.at[idx_v], rows_v, sem).wait()
            pltpu.sync_copy(rows_v, out_hbm.at[pl.ds(base + j * K, K)])
            return carry

        lax.fori_loop(0, CH, body, 0, unroll=False)

    return gather_kernel(x, col2)


def _sc_scatter(msg, row2):
    mesh = plsc.VectorSubcoreMesh(core_axis_name="c", subcore_axis_name="s")

    @functools.partial(
        pl.kernel,
        out_type=jax.ShapeDtypeStruct((NC, N, D), jnp.float32),
        mesh=mesh,
        scratch_types=[
            pltpu.VMEM((K,), jnp.int32),
            pltpu.VMEM((K, D), jnp.float32),
            pltpu.VMEM((ZR, D), jnp.float32),
            pltpu.VMEM_SHARED((N, D), jnp.float32),
            pltpu.SemaphoreType.DMA,
        ],
    )
    def scatter_kernel(msg_hbm, row_hbm, out_hbm, idx_v, msg_v, zbuf_v,
                       aggr_sh, sem):
        c = lax.axis_index("c")
        s = lax.axis_index("s")
        wid = c * NS + s
        base = wid * EPW

        zbuf_v[...] = jnp.zeros((ZR, D), jnp.float32)

        def zbody(t, carry):
            chunk = s + t * NS

            @pl.when(chunk < NZCH)
            def _():
                pltpu.sync_copy(zbuf_v, aggr_sh.at[pl.ds(chunk * ZR, ZR)])

            return carry

        lax.fori_loop(0, (NZCH + NS - 1) // NS, zbody, 0, unroll=False)
        plsc.subcore_barrier()

        def body(j, carry):
            pltpu.sync_copy(row_hbm.at[wid * CH + j], idx_v)
            pltpu.sync_copy(msg_hbm.at[pl.ds(base + j * K, K)], msg_v)
            pltpu.sync_copy(msg_v, aggr_sh.at[idx_v], add=True)
            return carry

        lax.fori_loop(0, CH, body, 0, unroll=False)
        plsc.subcore_barrier()

        def dbody(t, carry):
            chunk = s + t * NS

            @pl.when(chunk < NZCH)
            def _():
                pltpu.sync_copy(aggr_sh.at[pl.ds(chunk * ZR, ZR)],
                                out_hbm.at[c, pl.ds(chunk * ZR, ZR)])

            return carry

        lax.fori_loop(0, (NZCH + NS - 1) // NS, dbody, 0, unroll=False)

    return scatter_kernel(msg, row2)


def _edge_body(xg_ref, ea_ref, nW1_ref, nb1_ref, nW2_ref, nb2_ref,
               eW1_ref, eb1_ref, eW2_ref, eb2_ref, mW_ref, mb_ref,
               gW1e_ref, gW1x_ref, gb1_ref, gW2_ref, gb2_ref, msg_ref):
    bf = jnp.bfloat16
    f32 = jnp.float32
    xg = xg_ref[...].astype(bf)
    ea = ea_ref[...].astype(bf)
    h1 = jnp.maximum(
        jnp.dot(xg, nW1_ref[...], preferred_element_type=f32) + nb1_ref[...], 0.0)
    hn = jnp.dot(h1.astype(bf), nW2_ref[...], preferred_element_type=f32) + nb2_ref[...]
    e1 = jnp.maximum(
        jnp.dot(ea, eW1_ref[...], preferred_element_type=f32) + eb1_ref[...], 0.0)
    he = jnp.dot(e1.astype(bf), eW2_ref[...], preferred_element_type=f32) + eb2_ref[...]
    m = jnp.dot((he * hn).astype(bf), mW_ref[...],
                preferred_element_type=f32) + mb_ref[...]
    g1 = jnp.maximum(
        jnp.dot(ea, gW1e_ref[...], preferred_element_type=f32)
        + jnp.dot(xg, gW1x_ref[...], preferred_element_type=f32)
        + gb1_ref[...], 0.0)
    gt = jnp.dot(g1.astype(bf), gW2_ref[...], preferred_element_type=f32) + gb2_ref[...]
    msg_ref[...] = m * (1.0 / (1.0 + jnp.exp(-gt)))


def _tc_edge(xg, ea, nW1, nb1, nW2, nb2, eW1, eb1, eW2, eb2, mW, mb,
             gW1e, gW1x, gb1, gW2, gb2):
    grid = (E // BE,)
    eb = lambda i: (i, 0)
    wb = lambda i: (0, 0)
    full = lambda shape: pl.BlockSpec(shape, wb)
    return pl.pallas_call(
        _edge_body,
        grid=grid,
        in_specs=[
            pl.BlockSpec((BE, D), eb),
            pl.BlockSpec((BE, EA), eb),
            full((D, D)), full((1, D)), full((D, D)), full((1, D)),
            full((EA, D)), full((1, D)), full((D, D)), full((1, D)),
            full((D, D)), full((1, D)),
            full((EA, D)), full((D, D)), full((1, D)), full((D, D)),
            full((1, D)),
        ],
        out_specs=pl.BlockSpec((BE, D), eb),
        out_shape=jax.ShapeDtypeStruct((E, D), jnp.float32),
    )(xg, ea, nW1, nb1, nW2, nb2, eW1, eb1, eW2, eb2, mW, mb,
      gW1e, gW1x, gb1, gW2, gb2)


def _node_body(x_ref, p0_ref, p1_ref, cW_ref, cb_ref, g_ref, b_ref,
               oW_ref, ob_ref, out_ref):
    f32 = jnp.float32
    o = (jnp.dot(x_ref[...], cW_ref[...], preferred_element_type=f32)
         + cb_ref[...] + p0_ref[...] + p1_ref[...])
    mu = jnp.mean(o, axis=-1, keepdims=True)
    var = jnp.mean((o - mu) * (o - mu), axis=-1, keepdims=True)
    o = (o - mu) / jnp.sqrt(var + 1e-5) * g_ref[...] + b_ref[...]
    out_ref[...] = (jnp.dot(jnp.maximum(o, 0.0), oW_ref[...],
                            preferred_element_type=f32) + ob_ref[...])


def _tc_node(x, p0, p1, cW, cb, ln_g, ln_b, oW, ob):
    grid = (N // BN,)
    nb = lambda i: (i, 0)
    wb = lambda i: (0, 0)
    full = lambda shape: pl.BlockSpec(shape, wb)
    return pl.pallas_call(
        _node_body,
        grid=grid,
        in_specs=[
            pl.BlockSpec((BN, D), nb),
            pl.BlockSpec((BN, D), nb),
            pl.BlockSpec((BN, D), nb),
            full((D, D)), full((1, D)), full((1, D)), full((1, D)),
            full((D, D)), full((1, D)),
        ],
        out_specs=pl.BlockSpec((BN, D), nb),
        out_shape=jax.ShapeDtypeStruct((N, D), jnp.float32),
    )(x, p0, p1, cW, cb, ln_g, ln_b, oW, ob)


def kernel(x, pos, edge_index, edge_attr, node_W1, node_b1, node_W2, node_b2,
           edge_W1, edge_b1, edge_W2, edge_b2, msg_W, msg_b,
           gate_W1, gate_b1, gate_W2, gate_b2,
           cent_W, cent_b, ln_g, ln_b, out_W, out_b):
    row = edge_index[0].astype(jnp.int32)
    col = edge_index[1].astype(jnp.int32)
    col2 = col.reshape(NW * CH, K)
    row2 = row.reshape(NW * CH, K)

    r = lambda v: v.reshape(1, D)

    bf = lambda v: v.astype(jnp.bfloat16)

    xg = _sc_gather(x, col2)
    msg = _tc_edge(xg, edge_attr,
                   bf(node_W1), r(node_b1), bf(node_W2), r(node_b2),
                   bf(edge_W1), r(edge_b1), bf(edge_W2), r(edge_b2),
                   bf(msg_W), r(msg_b),
                   bf(gate_W1[:EA]), bf(gate_W1[EA:]), r(gate_b1),
                   bf(gate_W2), r(gate_b2))
    partials = _sc_scatter(msg, row2)
    out = _tc_node(x, partials[0], partials[1],
                   cent_W, r(cent_b), r(ln_g), r(ln_b), out_W, r(out_b))
    return out


# speedup vs baseline: 1.5802x; 1.5802x over previous
import functools

import jax
import jax.numpy as jnp
from jax import lax
from jax.experimental import pallas as pl
from jax.experimental.pallas import tpu as pltpu
from jax.experimental.pallas import tpu_sc as plsc

N = 10000
E = 320000
D = 128
DP = D // 2
EA = 16

NC = 2
NS = 16
NW = NC * NS
EPW = E // NW
K = 80
CH = EPW // K
NB = 5
NT = CH // NB

ZR = 40
NZCH = N // ZR

BE = 2000
BN = 1000


def _sc_gather(xp, col3):
    mesh = plsc.VectorSubcoreMesh(core_axis_name="c", subcore_axis_name="s")

    @functools.partial(
        pl.kernel,
        out_type=jax.ShapeDtypeStruct((E, D), jnp.int32),
        mesh=mesh,
        scratch_types=[
            pltpu.VMEM((CH, K), jnp.int32),
            [pltpu.VMEM((K, D), jnp.int32) for _ in range(NB)],
            [pltpu.SemaphoreType.DMA for _ in range(NB)],
            [pltpu.SemaphoreType.DMA for _ in range(NB)],
        ],
    )
    def gather_kernel(x_hbm, col_hbm, out_hbm, idx_all, bufs, gsems, wsems):
        wid = lax.axis_index("c") * NS + lax.axis_index("s")
        base = wid * EPW

        pltpu.sync_copy(col_hbm.at[wid], idx_all)
        for b in range(NB):
            pltpu.async_copy(x_hbm.at[idx_all.at[b]], bufs[b], gsems[b])

        def body(t, carry):
            for b in range(NB):
                ch = t * NB + b
                pltpu.make_async_copy(x_hbm.at[idx_all.at[ch]], bufs[b],
                                      gsems[b]).wait()
                pltpu.async_copy(bufs[b],
                                 out_hbm.at[pl.ds(base + ch * K, K)],
                                 wsems[b])
            for b in range(NB):
                nch = (t + 1) * NB + b

                @pl.when(t + 1 < NT)
                def _():
                    pltpu.make_async_copy(
                        bufs[b], out_hbm.at[pl.ds(base, K)], wsems[b]).wait()
                    pltpu.async_copy(x_hbm.at[idx_all.at[nch]], bufs[b],
                                     gsems[b])

            return carry

        lax.fori_loop(0, NT, body, 0, unroll=False)
        for b in range(NB):
            pltpu.make_async_copy(
                bufs[b], out_hbm.at[pl.ds(base, K)], wsems[b]).wait()

    return gather_kernel(xp, col3)


KS = 40
CHS = EPW // KS
NBS = 5
NTS = CHS // NBS


def _sc_scatter(msg, row3):
    mesh = plsc.VectorSubcoreMesh(core_axis_name="c", subcore_axis_name="s")

    @functools.partial(
        pl.kernel,
        out_type=jax.ShapeDtypeStruct((NC, N, D), jnp.float32),
        mesh=mesh,
        scratch_types=[
            [pltpu.VMEM((KS,), jnp.int32) for _ in range(NBS)],
            [pltpu.VMEM((KS, D), jnp.float32) for _ in range(NBS)],
            [pltpu.SemaphoreType.DMA for _ in range(NBS)],
            [pltpu.SemaphoreType.DMA for _ in range(NBS)],
            [pltpu.SemaphoreType.DMA for _ in range(NBS)],
            pltpu.VMEM_SHARED((N, D), jnp.float32),
        ],
    )
    def scatter_kernel(msg_hbm, row_hbm, out_hbm, ibufs, mbufs, isems,
                       msems, ssems, aggr_sh):
        c = lax.axis_index("c")
        s = lax.axis_index("s")
        wid = c * NS + s
        base = wid * EPW

        mbufs[0][...] = jnp.zeros((KS, D), jnp.float32)

        def zbody(t, carry):
            chunk = s + t * NS

            @pl.when(chunk < N // KS)
            def _():
                pltpu.sync_copy(mbufs[0], aggr_sh.at[pl.ds(chunk * KS, KS)])

            return carry

        lax.fori_loop(0, (N // KS + NS - 1) // NS, zbody, 0, unroll=False)
        plsc.subcore_barrier()

        for b in range(NBS):
            pltpu.async_copy(row_hbm.at[wid, b], ibufs[b], isems[b])
            pltpu.async_copy(msg_hbm.at[pl.ds(base + b * KS, KS)], mbufs[b],
                             msems[b])

        def body(t, carry):
            for b in range(NBS):
                ch = t * NBS + b
                pltpu.make_async_copy(row_hbm.at[wid, ch], ibufs[b],
                                      isems[b]).wait()
                pltpu.make_async_copy(
                    msg_hbm.at[pl.ds(base + ch * KS, KS)], mbufs[b],
                    msems[b]).wait()
                pltpu.async_copy(mbufs[b], aggr_sh.at[ibufs[b]],
                                 ssems[b], add=True)
            for b in range(NBS):
                nch = (t + 1) * NBS + b

                @pl.when(t + 1 < NTS)
                def _():
                    pltpu.make_async_copy(mbufs[b], aggr_sh.at[ibufs[b]],
                                          ssems[b]).wait()
                    pltpu.async_copy(row_hbm.at[wid, nch], ibufs[b],
                                     isems[b])
                    pltpu.async_copy(
                        msg_hbm.at[pl.ds(base + nch * KS, KS)], mbufs[b],
                        msems[b])

            return carry

        lax.fori_loop(0, NTS, body, 0, unroll=False)
        for b in range(NBS):
            pltpu.make_async_copy(mbufs[b], aggr_sh.at[ibufs[b]],
                                  ssems[b]).wait()
        plsc.subcore_barrier()

        def dbody(t, carry):
            chunk = s + t * NS

            @pl.when(chunk < N // KS)
            def _():
                pltpu.sync_copy(aggr_sh.at[pl.ds(chunk * KS, KS)],
                                out_hbm.at[c, pl.ds(chunk * KS, KS)])

            return carry

        lax.fori_loop(0, (N // KS + NS - 1) // NS, dbody, 0, unroll=False)

    return scatter_kernel(msg, row3)


def _hnode_body(x_ref, nW1_ref, nb1_ref, nW2_ref, nb2_ref, out_ref):
    bf = jnp.bfloat16
    f32 = jnp.float32
    u16 = jnp.uint16
    x = x_ref[...]
    h1 = jnp.maximum(
        jnp.dot(x.astype(bf), nW1_ref[...], preferred_element_type=f32)
        + nb1_ref[...], 0.0)
    hn = (jnp.dot(h1.astype(bf), nW2_ref[...], preferred_element_type=f32)
          + nb2_ref[...])
    xb = lax.bitcast_convert_type(x.astype(bf), u16).astype(jnp.int32)
    hb = lax.bitcast_convert_type(hn.astype(bf), u16).astype(jnp.int32)
    out_ref[...] = jnp.bitwise_or(lax.shift_left(hb, 16), xb)


def _tc_hnode(x, nW1, nb1, nW2, nb2):
    grid = (N // BN,)
    nb = lambda i: (i, 0)
    wb = lambda i: (0, 0)
    full = lambda shape: pl.BlockSpec(shape, wb)
    return pl.pallas_call(
        _hnode_body,
        grid=grid,
        in_specs=[
            pl.BlockSpec((BN, D), nb),
            full((D, D)), full((1, D)), full((D, D)), full((1, D)),
        ],
        out_specs=pl.BlockSpec((BN, D), nb),
        out_shape=jax.ShapeDtypeStruct((N, D), jnp.int32),
    )(x, nW1, nb1, nW2, nb2)


def _edge_body(xg_ref, ea_ref,
               eW1_ref, eb1_ref, eW2_ref, eb2_ref, mW_ref, mb_ref,
               gW1e_ref, gW1x_ref, gb1_ref, gW2_ref, gb2_ref, msg_ref):
    bf = jnp.bfloat16
    f32 = jnp.float32
    xi = xg_ref[...]
    xg = lax.bitcast_convert_type(lax.shift_left(xi, 16), f32).astype(bf)
    hn = lax.bitcast_convert_type(
        jnp.bitwise_and(xi, jnp.int32(-65536)), f32)
    ea = ea_ref[...].astype(bf)
    e1 = jnp.maximum(
        jnp.dot(ea, eW1_ref[...], preferred_element_type=f32) + eb1_ref[...], 0.0)
    he = jnp.dot(e1.astype(bf), eW2_ref[...], preferred_element_type=f32) + eb2_ref[...]
    m = jnp.dot((he * hn).astype(bf), mW_ref[...],
                preferred_element_type=f32) + mb_ref[...]
    g1 = jnp.maximum(
        jnp.dot(ea, gW1e_ref[...], preferred_element_type=f32)
        + jnp.dot(xg, gW1x_ref[...], preferred_element_type=f32)
        + gb1_ref[...], 0.0)
    gt = jnp.dot(g1.astype(bf), gW2_ref[...], preferred_element_type=f32) + gb2_ref[...]
    msg_ref[...] = m * (1.0 / (1.0 + jnp.exp(-gt)))


def _tc_edge(xgp, ea, eW1, eb1, eW2, eb2, mW, mb,
             gW1e, gW1x, gb1, gW2, gb2):
    grid = (E // BE,)
    eb = lambda i: (i, 0)
    wb = lambda i: (0, 0)
    full = lambda shape: pl.BlockSpec(shape, wb)
    return pl.pallas_call(
        _edge_body,
        grid=grid,
        in_specs=[
            pl.BlockSpec((BE, D), eb),
            pl.BlockSpec((BE, EA), eb),
            full((EA, D)), full((1, D)), full((D, D)), full((1, D)),
            full((D, D)), full((1, D)),
            full((EA, D)), full((D, D)), full((1, D)), full((D, D)),
            full((1, D)),
        ],
        out_specs=pl.BlockSpec((BE, D), eb),
        out_shape=jax.ShapeDtypeStruct((E, D), jnp.float32),
    )(xgp, ea, eW1, eb1, eW2, eb2, mW, mb,
      gW1e, gW1x, gb1, gW2, gb2)


def _node_body(x_ref, p0_ref, p1_ref, cW_ref, cb_ref, g_ref, b_ref,
               oW_ref, ob_ref, out_ref):
    f32 = jnp.float32
    o = (jnp.dot(x_ref[...], cW_ref[...], preferred_element_type=f32)
         + cb_ref[...] + p0_ref[...] + p1_ref[...])
    mu = jnp.mean(o, axis=-1, keepdims=True)
    var = jnp.mean((o - mu) * (o - mu), axis=-1, keepdims=True)
    o = (o - mu) / jnp.sqrt(var + 1e-5) * g_ref[...] + b_ref[...]
    out_ref[...] = (jnp.dot(jnp.maximum(o, 0.0), oW_ref[...],
                            preferred_element_type=f32) + ob_ref[...])


def _tc_node(x, p0, p1, cW, cb, ln_g, ln_b, oW, ob):
    grid = (N // BN,)
    nb = lambda i: (i, 0)
    wb = lambda i: (0, 0)
    full = lambda shape: pl.BlockSpec(shape, wb)
    return pl.pallas_call(
        _node_body,
        grid=grid,
        in_specs=[
            pl.BlockSpec((BN, D), nb),
            pl.BlockSpec((BN, D), nb),
            pl.BlockSpec((BN, D), nb),
            full((D, D)), full((1, D)), full((1, D)), full((1, D)),
            full((D, D)), full((1, D)),
        ],
        out_specs=pl.BlockSpec((BN, D), nb),
        out_shape=jax.ShapeDtypeStruct((N, D), jnp.float32),
    )(x, p0, p1, cW, cb, ln_g, ln_b, oW, ob)


def kernel(x, pos, edge_index, edge_attr, node_W1, node_b1, node_W2, node_b2,
           edge_W1, edge_b1, edge_W2, edge_b2, msg_W, msg_b,
           gate_W1, gate_b1, gate_W2, gate_b2,
           cent_W, cent_b, ln_g, ln_b, out_W, out_b):
    row = edge_index[0].astype(jnp.int32)
    col = edge_index[1].astype(jnp.int32)
    col3 = col.reshape(NW, CH, K)
    row3 = row.reshape(NW, CHS, KS)

    r = lambda v: v.reshape(1, D)
    bf = lambda v: v.astype(jnp.bfloat16)

    xp = _tc_hnode(x, bf(node_W1), r(node_b1), bf(node_W2), r(node_b2))

    xgp = _sc_gather(xp, col3)
    msg = _tc_edge(xgp, edge_attr,
                   bf(edge_W1), r(edge_b1), bf(edge_W2), r(edge_b2),
                   bf(msg_W), r(msg_b),
                   bf(gate_W1[:EA]), bf(gate_W1[EA:]), r(gate_b1),
                   bf(gate_W2), r(gate_b2))
    partials = _sc_scatter(msg, row3)
    out = _tc_node(x, partials[0], partials[1],
                   cent_W, r(cent_b), r(ln_g), r(ln_b), out_W, r(out_b))
    return out
